# Initial kernel scaffold; baseline (speedup 1.0000x reference)
import jax
import jax.numpy as jnp
from jax.experimental import pallas as pl


def kernel(x, edge_index, conv1d_w, conv1d_b, W_self1, W_neigh1, b1, W_self2, W_neigh2, b2, fc1_w, fc1_b, fc2_w, fc2_b, fc3_w, fc3_b):
    raise NotImplementedError("write your pallas kernel here")


# speedup vs baseline: 6.4919x; 6.4919x over previous
import functools

import jax
import jax.numpy as jnp
from jax import lax
from jax.experimental import pallas as pl
from jax.experimental.pallas import tpu as pltpu
from jax.experimental.pallas import tpu_sc as plsc

N = 10000
E = 320000
D_IN = 128
KSZ = 10
D_CONV = D_IN - (KSZ - 1)
H = 64

NC = 2
NS = 16
EPT = E // (NC * NS)
CHUNK = 80
NCHUNK = EPT // CHUNK
N_PAD = 10240
RPT = N_PAD // NS
RCHUNK = 128
D_AUG = 80


@functools.lru_cache(maxsize=None)
def _make_segsum(D):
  mesh = plsc.VectorSubcoreMesh(
      core_axis_name="c", subcore_axis_name="s", num_cores=NC, num_subcores=NS)

  @functools.partial(
      pl.kernel,
      out_type=jax.ShapeDtypeStruct((NC * N_PAD, D), jnp.float32),
      mesh=mesh,
      compiler_params=pltpu.CompilerParams(use_tc_tiling_on_sc=False),
      scratch_types=[
          pltpu.VMEM((CHUNK,), jnp.int32),
          pltpu.VMEM((CHUNK,), jnp.int32),
          pltpu.VMEM((CHUNK, D), jnp.float32),
          pltpu.VMEM((RCHUNK, D), jnp.float32),
          pltpu.VMEM_SHARED((N_PAD, D), jnp.float32),
          pltpu.SemaphoreType.DMA,
      ],
  )
  def seg(src_hbm, dst_hbm, table_hbm, out_hbm, sidx, didx, rows, zbuf, agg, sem):
    c = lax.axis_index("c")
    s = lax.axis_index("s")
    kd = D // 16

    zv = jnp.zeros((16,), jnp.float32)

    def zfill(i, _):
      for k in range(kd):
        zbuf[i, pl.ds(k * 16, 16)] = zv
      return _

    lax.fori_loop(0, RCHUNK, zfill, None)

    row0 = s * RPT

    def zinit(j, _):
      pltpu.sync_copy(zbuf, agg.at[pl.ds(row0 + j * RCHUNK, RCHUNK)])
      return _

    lax.fori_loop(0, RPT // RCHUNK, zinit, None)
    plsc.subcore_barrier()

    ebase = (c * NS + s) * EPT

    def body(i, _):
      e0 = ebase + i * CHUNK
      pltpu.sync_copy(src_hbm.at[pl.ds(e0, CHUNK)], sidx)
      pltpu.sync_copy(dst_hbm.at[pl.ds(e0, CHUNK)], didx)
      pltpu.async_copy(table_hbm.at[sidx], rows, sem).wait()
      pltpu.sync_copy(rows, agg.at[didx], add=True)
      return _

    lax.fori_loop(0, NCHUNK, body, None)
    plsc.subcore_barrier()

    obase = c * N_PAD + row0

    def wout(j, _):
      pltpu.sync_copy(agg.at[pl.ds(row0 + j * RCHUNK, RCHUNK)], zbuf)
      pltpu.sync_copy(zbuf, out_hbm.at[pl.ds(obase + j * RCHUNK, RCHUNK)])
      return _

    lax.fori_loop(0, RPT // RCHUNK, wout, None)

  return seg


def _prologue_body(w_ref, b_ref, x_ref, ws_ref, wn_ref, hs_ref, taug_ref):
  d_i = lax.broadcasted_iota(jnp.int32, (D_IN, D_CONV), 0)
  j_i = lax.broadcasted_iota(jnp.int32, (D_IN, D_CONV), 1)
  k = d_i - j_i
  wc = jnp.zeros((D_IN, D_CONV), jnp.float32)
  for t in range(KSZ):
    wc = wc + jnp.where(k == t, w_ref[0, t], 0.0)
  h0 = jnp.dot(x_ref[...], wc, preferred_element_type=jnp.float32) + b_ref[0, 0]
  hs_ref[...] = jnp.dot(h0, ws_ref[...], preferred_element_type=jnp.float32)
  hw = jnp.dot(h0, wn_ref[...], preferred_element_type=jnp.float32)
  taug_ref[:, :H] = hw
  col = lax.broadcasted_iota(jnp.int32, (N, D_AUG - H), 1)
  taug_ref[:, H:] = jnp.where(col == 0, 1.0, 0.0)


_prologue = pl.pallas_call(
    _prologue_body,
    out_shape=[
        jax.ShapeDtypeStruct((N, H), jnp.float32),
        jax.ShapeDtypeStruct((N, D_AUG), jnp.float32),
    ],
    in_specs=[
        pl.BlockSpec(memory_space=pltpu.SMEM),
        pl.BlockSpec(memory_space=pltpu.SMEM),
        pl.BlockSpec(),
        pl.BlockSpec(),
        pl.BlockSpec(),
    ],
)


def _mid_body(hs1_ref, p0_ref, p1_ref, b1_ref, ws2_ref, wn2_ref,
              hs2_ref, hw2_ref, invdeg_ref):
  p = p0_ref[...] + p1_ref[...]
  agg = p[:, :H]
  deg = p[:, H:H + 1]
  invdeg = 1.0 / jnp.maximum(deg, 1.0)
  h1 = jax.nn.relu(hs1_ref[...] + agg * invdeg + b1_ref[...])
  hs2_ref[...] = jnp.dot(h1, ws2_ref[...], preferred_element_type=jnp.float32)
  hw2_ref[...] = jnp.dot(h1, wn2_ref[...], preferred_element_type=jnp.float32)
  invdeg_ref[...] = invdeg


_mid = pl.pallas_call(
    _mid_body,
    out_shape=[
        jax.ShapeDtypeStruct((N, H // 2), jnp.float32),
        jax.ShapeDtypeStruct((N, H // 2), jnp.float32),
        jax.ShapeDtypeStruct((N, 1), jnp.float32),
    ],
)


def _tail_body(hs2_ref, q0_ref, q1_ref, invdeg_ref, b2_ref,
               w1_ref, b1_ref, w2_ref, b2f_ref, w3_ref, b3_ref, out_ref):
  agg = q0_ref[...] + q1_ref[...]
  h2 = jax.nn.relu(hs2_ref[...] + agg * invdeg_ref[...] + b2_ref[...])
  a = jax.nn.relu(
      jnp.dot(h2, w1_ref[...], preferred_element_type=jnp.float32) + b1_ref[...])
  b = jax.nn.relu(
      jnp.dot(a, w2_ref[...], preferred_element_type=jnp.float32) + b2f_ref[...])
  out_ref[...] = jnp.sum(b * w3_ref[...], axis=1, keepdims=True) + b3_ref[...]


_tail = pl.pallas_call(
    _tail_body,
    out_shape=jax.ShapeDtypeStruct((N, 1), jnp.float32),
)


def kernel(x, edge_index, conv1d_w, conv1d_b, W_self1, W_neigh1, b1,
           W_self2, W_neigh2, b2, fc1_w, fc1_b, fc2_w, fc2_b, fc3_w, fc3_b):
  hs1, taug = _prologue(conv1d_w.reshape(1, KSZ), conv1d_b.reshape(1, 1),
                        x, W_self1, W_neigh1)
  src = edge_index[0]
  dst = edge_index[1]
  p = _make_segsum(D_AUG)(src, dst, taug)
  hs2, hw2, invdeg = _mid(hs1, p[:N], p[N_PAD:N_PAD + N], b1.reshape(1, H),
                          W_self2, W_neigh2)
  q = _make_segsum(H // 2)(src, dst, hw2)
  out = _tail(hs2, q[:N], q[N_PAD:N_PAD + N], invdeg, b2.reshape(1, H // 2),
              fc1_w, fc1_b.reshape(1, H // 4), fc2_w, fc2_b.reshape(1, H // 8),
              fc3_w.reshape(1, H // 8), fc3_b.reshape(1, 1))
  return out


# speedup vs baseline: 17.7584x; 2.7355x over previous
import functools

import jax
import jax.numpy as jnp
from jax import lax
from jax.experimental import pallas as pl
from jax.experimental.pallas import tpu as pltpu
from jax.experimental.pallas import tpu_sc as plsc

N = 10000
E = 320000
D_IN = 128
KSZ = 10
D_CONV = D_IN - (KSZ - 1)
H = 64

NC = 2
NS = 16
EPT = E // (NC * NS)
CHUNK = 80
NCHUNK = EPT // CHUNK
N_PAD = 10240
RPT = N_PAD // NS
RCHUNK = 128
D_AUG = 80


NBUF = 5
NGRP = NCHUNK // NBUF


@functools.lru_cache(maxsize=None)
def _make_segsum(D):
  mesh = plsc.VectorSubcoreMesh(
      core_axis_name="c", subcore_axis_name="s", num_cores=NC, num_subcores=NS)

  @functools.partial(
      pl.kernel,
      out_type=jax.ShapeDtypeStruct((NC * N_PAD, D), jnp.float32),
      mesh=mesh,
      compiler_params=pltpu.CompilerParams(use_tc_tiling_on_sc=False),
      scratch_types=[
          pltpu.VMEM((NCHUNK, CHUNK), jnp.int32),
          pltpu.VMEM((NCHUNK, CHUNK), jnp.int32),
          [pltpu.VMEM((CHUNK, D), jnp.float32) for _ in range(NBUF)],
          pltpu.VMEM((RCHUNK, D), jnp.float32),
          pltpu.VMEM_SHARED((N_PAD, D), jnp.float32),
          pltpu.SemaphoreType.DMA,
          [pltpu.SemaphoreType.DMA for _ in range(NBUF)],
          [pltpu.SemaphoreType.DMA for _ in range(NBUF)],
      ],
  )
  def seg(src_hbm, dst_hbm, table_hbm, out_hbm,
          sidx, didx, rows, zbuf, agg, sem_i, sem_g, sem_s):
    c = lax.axis_index("c")
    s = lax.axis_index("s")
    tid = c * NS + s
    kd = D // 16

    di_s = pltpu.async_copy(src_hbm.at[tid], sidx, sem_i)
    di_d = pltpu.async_copy(dst_hbm.at[tid], didx, sem_i)

    zv = jnp.zeros((16,), jnp.float32)

    def zfill(i, _):
      for k in range(kd):
        zbuf[i, pl.ds(k * 16, 16)] = zv
      return _

    lax.fori_loop(0, RCHUNK, zfill, None)

    row0 = s * RPT

    def zinit(j, _):
      pltpu.sync_copy(zbuf, agg.at[pl.ds(row0 + j * RCHUNK, RCHUNK)])
      return _

    lax.fori_loop(0, RPT // RCHUNK, zinit, None)
    di_s.wait()
    di_d.wait()
    plsc.subcore_barrier()

    def turn(g, _):
      gd = []
      for b in range(NBUF):
        @pl.when(g > 0)
        def _wait_prev():
          pltpu.make_async_copy(
              rows[b], agg.at[didx.at[g * NBUF + b]], sem_s[b]).wait()
        gd.append(pltpu.async_copy(
            table_hbm.at[sidx.at[g * NBUF + b]], rows[b], sem_g[b]))
      for b in range(NBUF):
        gd[b].wait()
        pltpu.async_copy(
            rows[b], agg.at[didx.at[g * NBUF + b]], sem_s[b], add=True)
      return _

    lax.fori_loop(0, NGRP, turn, None)
    for b in range(NBUF):
      pltpu.make_async_copy(
          rows[b], agg.at[didx.at[b]], sem_s[b]).wait()
    plsc.subcore_barrier()

    obase = c * N_PAD + row0

    def wout(j, _):
      pltpu.sync_copy(agg.at[pl.ds(row0 + j * RCHUNK, RCHUNK)], zbuf)
      pltpu.sync_copy(zbuf, out_hbm.at[pl.ds(obase + j * RCHUNK, RCHUNK)])
      return _

    lax.fori_loop(0, RPT // RCHUNK, wout, None)

  return seg


def _prologue_body(w_ref, b_ref, x_ref, ws_ref, wn_ref, hs_ref, taug_ref):
  d_i = lax.broadcasted_iota(jnp.int32, (D_IN, D_CONV), 0)
  j_i = lax.broadcasted_iota(jnp.int32, (D_IN, D_CONV), 1)
  k = d_i - j_i
  wc = jnp.zeros((D_IN, D_CONV), jnp.float32)
  for t in range(KSZ):
    wc = wc + jnp.where(k == t, w_ref[0, t], 0.0)
  h0 = jnp.dot(x_ref[...], wc, preferred_element_type=jnp.float32) + b_ref[0, 0]
  hs_ref[...] = jnp.dot(h0, ws_ref[...], preferred_element_type=jnp.float32)
  hw = jnp.dot(h0, wn_ref[...], preferred_element_type=jnp.float32)
  taug_ref[:, :H] = hw
  col = lax.broadcasted_iota(jnp.int32, (N, D_AUG - H), 1)
  taug_ref[:, H:] = jnp.where(col == 0, 1.0, 0.0)


_prologue = pl.pallas_call(
    _prologue_body,
    out_shape=[
        jax.ShapeDtypeStruct((N, H), jnp.float32),
        jax.ShapeDtypeStruct((N, D_AUG), jnp.float32),
    ],
    in_specs=[
        pl.BlockSpec(memory_space=pltpu.SMEM),
        pl.BlockSpec(memory_space=pltpu.SMEM),
        pl.BlockSpec(),
        pl.BlockSpec(),
        pl.BlockSpec(),
    ],
)


def _mid_body(hs1_ref, p0_ref, p1_ref, b1_ref, ws2_ref, wn2_ref,
              hs2_ref, hw2_ref, invdeg_ref):
  p = p0_ref[...] + p1_ref[...]
  agg = p[:, :H]
  deg = p[:, H:H + 1]
  invdeg = 1.0 / jnp.maximum(deg, 1.0)
  h1 = jax.nn.relu(hs1_ref[...] + agg * invdeg + b1_ref[...])
  hs2_ref[...] = jnp.dot(h1, ws2_ref[...], preferred_element_type=jnp.float32)
  hw2_ref[...] = jnp.dot(h1, wn2_ref[...], preferred_element_type=jnp.float32)
  invdeg_ref[...] = invdeg


_mid = pl.pallas_call(
    _mid_body,
    out_shape=[
        jax.ShapeDtypeStruct((N, H // 2), jnp.float32),
        jax.ShapeDtypeStruct((N, H // 2), jnp.float32),
        jax.ShapeDtypeStruct((N, 1), jnp.float32),
    ],
)


def _tail_body(hs2_ref, q0_ref, q1_ref, invdeg_ref, b2_ref,
               w1_ref, b1_ref, w2_ref, b2f_ref, w3_ref, b3_ref, out_ref):
  agg = q0_ref[...] + q1_ref[...]
  h2 = jax.nn.relu(hs2_ref[...] + agg * invdeg_ref[...] + b2_ref[...])
  a = jax.nn.relu(
      jnp.dot(h2, w1_ref[...], preferred_element_type=jnp.float32) + b1_ref[...])
  b = jax.nn.relu(
      jnp.dot(a, w2_ref[...], preferred_element_type=jnp.float32) + b2f_ref[...])
  out_ref[...] = jnp.sum(b * w3_ref[...], axis=1, keepdims=True) + b3_ref[...]


_tail = pl.pallas_call(
    _tail_body,
    out_shape=jax.ShapeDtypeStruct((N, 1), jnp.float32),
)


def kernel(x, edge_index, conv1d_w, conv1d_b, W_self1, W_neigh1, b1,
           W_self2, W_neigh2, b2, fc1_w, fc1_b, fc2_w, fc2_b, fc3_w, fc3_b):
  hs1, taug = _prologue(conv1d_w.reshape(1, KSZ), conv1d_b.reshape(1, 1),
                        x, W_self1, W_neigh1)
  src = edge_index[0].reshape(NC * NS, NCHUNK, CHUNK)
  dst = edge_index[1].reshape(NC * NS, NCHUNK, CHUNK)
  p = _make_segsum(D_AUG)(src, dst, taug)
  hs2, hw2, invdeg = _mid(hs1, p[:N], p[N_PAD:N_PAD + N], b1.reshape(1, H),
                          W_self2, W_neigh2)
  q = _make_segsum(H // 2)(src, dst, hw2)
  out = _tail(hs2, q[:N], q[N_PAD:N_PAD + N], invdeg, b2.reshape(1, H // 2),
              fc1_w, fc1_b.reshape(1, H // 4), fc2_w, fc2_b.reshape(1, H // 8),
              fc3_w.reshape(1, H // 8), fc3_b.reshape(1, 1))
  return out
